# baseline (speedup 1.0000x reference)
import jax
import jax.numpy as jnp
from jax import lax
from jax.experimental import pallas as pl
from jax.experimental.pallas import tpu as pltpu
from jax.experimental.pallas import tpu_sc as plsc

L = 16
ROWS_PER_TILE = 512
IDX_PER_BATCH = 512
NCH = 32
NT = 32


def _route_body(vi_hbm, counts_hbm, rows_hbm, jrows_hbm,
                idx_v, winner, list_t, list_j, cnt_v):
    nc = 2
    wid = lax.axis_index("s") * nc + lax.axis_index("c")
    b = wid // 8
    seg = wid % 8
    seg_lo = seg * ROWS_PER_TILE

    pltpu.sync_copy(vi_hbm.at[pl.ds(b * IDX_PER_BATCH, IDX_PER_BATCH)], idx_v)

    iota = lax.iota(jnp.int32, L)

    for c in range(IDX_PER_BATCH // L):
        v = idx_v[pl.ds(c * L, L)]
        jl = iota + c * L
        valid = ((v >= seg_lo) & (v < seg_lo + ROWS_PER_TILE) & (v != 0))
        addr = jnp.clip(v - seg_lo, 0, ROWS_PER_TILE - 1)
        for lane in range(L):
            plsc.store_scatter(winner, [addr], jl,
                               mask=valid & (iota == lane))

    cnt = jnp.int32(0)
    for c in range(IDX_PER_BATCH // L):
        v = idx_v[pl.ds(c * L, L)]
        jl = iota + c * L
        valid0 = (v >= seg_lo) & (v < seg_lo + ROWS_PER_TILE) & (v != 0)
        addr = jnp.clip(v - seg_lo, 0, ROWS_PER_TILE - 1)
        w = plsc.load_gather(winner, [addr], mask=valid0)
        keep = valid0 & (w == jl)
        mi = keep.astype(jnp.int32)
        incl = plsc.cumsum(mi)
        pos = cnt + incl - mi
        plsc.store_scatter(list_t, [pos // L, pos % L], addr, mask=keep)
        plsc.store_scatter(list_j, [pos // L, pos % L],
                           b * IDX_PER_BATCH + jl, mask=keep)
        cnt = cnt + jnp.sum(mi)

    cnt_v[...] = jnp.broadcast_to(cnt, (L,)).astype(jnp.int32)
    pltpu.sync_copy(cnt_v, counts_hbm.at[wid])
    pltpu.sync_copy(list_t, rows_hbm.at[wid])
    pltpu.sync_copy(list_j, jrows_hbm.at[wid])


def _fold_body(counts_hbm, rows_hbm, jrows_hbm, img_hbm, t_ref, o_ref,
               smem_c, smem_r, smem_j, sem_l, sem_f):
    blk = pl.program_id(0)
    o_ref[...] = t_ref[...]
    cp_c = pltpu.make_async_copy(counts_hbm.at[blk], smem_c, sem_l)
    cp_r = pltpu.make_async_copy(rows_hbm.at[blk], smem_r, sem_l)
    cp_j = pltpu.make_async_copy(jrows_hbm.at[blk], smem_j, sem_l)
    cp_c.start(); cp_r.start(); cp_j.start()
    cp_c.wait(); cp_r.wait(); cp_j.wait()
    cnt = smem_c[0]

    def fire(i, carry):
        r = smem_r[i // L, i % L]
        j = smem_j[i // L, i % L]
        pltpu.make_async_copy(img_hbm.at[pl.ds(j, 1)],
                              o_ref.at[pl.ds(r, 1)], sem_f).start()
        return carry

    lax.fori_loop(0, cnt, fire, jnp.int32(0))

    def drain(i, carry):
        pltpu.make_async_copy(img_hbm.at[pl.ds(0, 1)],
                              o_ref.at[pl.ds(0, 1)], sem_f).wait()
        return carry

    lax.fori_loop(0, cnt, drain, jnp.int32(0))


@jax.jit
def _interleave(img_flat, text_flat, vi_flat):
    nrows, d = text_flat.shape

    mesh = plsc.VectorSubcoreMesh(core_axis_name="c", subcore_axis_name="s")
    counts, rows, jrows = pl.kernel(
        _route_body,
        out_type=(
            jax.ShapeDtypeStruct((NT, L), jnp.int32),
            jax.ShapeDtypeStruct((NT, NCH, L), jnp.int32),
            jax.ShapeDtypeStruct((NT, NCH, L), jnp.int32),
        ),
        mesh=mesh,
        scratch_types=[
            pltpu.VMEM((IDX_PER_BATCH,), jnp.int32),
            pltpu.VMEM((ROWS_PER_TILE,), jnp.int32),
            pltpu.VMEM((NCH, L), jnp.int32),
            pltpu.VMEM((NCH, L), jnp.int32),
            pltpu.VMEM((L,), jnp.int32),
        ],
        compiler_params=pltpu.CompilerParams(needs_layout_passes=False),
    )(vi_flat)

    hbm = pl.BlockSpec(memory_space=pltpu.MemorySpace.HBM)
    out = pl.pallas_call(
        _fold_body,
        grid=(NT,),
        in_specs=[hbm, hbm, hbm, hbm,
                  pl.BlockSpec((ROWS_PER_TILE, d), lambda i: (i, 0))],
        out_specs=pl.BlockSpec((ROWS_PER_TILE, d), lambda i: (i, 0)),
        out_shape=jax.ShapeDtypeStruct((nrows, d), text_flat.dtype),
        scratch_shapes=[
            pltpu.SMEM((L,), jnp.int32),
            pltpu.SMEM((NCH, L), jnp.int32),
            pltpu.SMEM((NCH, L), jnp.int32),
            pltpu.SemaphoreType.DMA,
            pltpu.SemaphoreType.DMA,
        ],
    )(counts, rows, jrows, img_flat, text_flat)
    return out


def kernel(image_embeddings, text_embeddings, vision_indices):
    B, S, D = text_embeddings.shape
    img_flat = image_embeddings.reshape(-1, D)
    text_flat = text_embeddings.reshape(B * S, D)
    vi_flat = vision_indices.astype(jnp.int32).reshape(-1)
    out = _interleave(img_flat, text_flat, vi_flat)
    return out.reshape(B, S, D)


# speedup vs baseline: 1.2333x; 1.2333x over previous
import jax
import jax.numpy as jnp
from jax import lax
from jax.experimental import pallas as pl
from jax.experimental.pallas import tpu as pltpu
from jax.experimental.pallas import tpu_sc as plsc
from jax._src.pallas import mpmd as pl_mpmd

L = 16
ROWS_PER_TILE = 512
IDX_PER_BATCH = 512
CHUNK = 16
NCH = IDX_PER_BATCH // CHUNK


def _copy_body(t_ref, o_ref):
    o_ref[...] = t_ref[...]


def _sc_body(img_hbm, copied_hbm, vi_hbm, out_hbm,
             idx_v, winner, list_t, list_j, buf_a, buf_b, buf_c,
             semg0, semg1, semg2, semp0, semp1, semp2):
    del copied_hbm
    nc = 2
    wid = lax.axis_index("s") * nc + lax.axis_index("c")
    b = wid // 8
    seg = wid % 8
    seg_lo = seg * ROWS_PER_TILE

    pltpu.sync_copy(vi_hbm.at[pl.ds(b * IDX_PER_BATCH, IDX_PER_BATCH)], idx_v)

    iota = lax.iota(jnp.int32, L)

    shift1 = jnp.minimum(iota + 1, L - 1)
    for c in range(IDX_PER_BATCH // L):
        v = idx_v[pl.ds(c * L, L)]
        jl = iota + c * L
        ks, js = plsc.sort_key_val(v * IDX_PER_BATCH + jl, jl)
        vs = ks // IDX_PER_BATCH
        nxt = vs.at[shift1].get(mode="promise_in_bounds")
        is_last = (vs != nxt) | (iota == L - 1)
        valid = ((vs >= seg_lo) & (vs < seg_lo + ROWS_PER_TILE)
                 & (vs != 0) & is_last)
        addr = jnp.clip(vs - seg_lo, 0, ROWS_PER_TILE - 1)
        plsc.store_scatter(winner, [addr], js, mask=valid)

    cnt = jnp.int32(0)
    for c in range(IDX_PER_BATCH // L):
        v = idx_v[pl.ds(c * L, L)]
        jl = iota + c * L
        valid0 = (v >= seg_lo) & (v < seg_lo + ROWS_PER_TILE) & (v != 0)
        addr = jnp.clip(v - seg_lo, 0, ROWS_PER_TILE - 1)
        w = plsc.load_gather(winner, [addr], mask=valid0)
        keep = valid0 & (w == jl)
        mi = keep.astype(jnp.int32)
        incl = plsc.cumsum(mi)
        pos = cnt + incl - mi
        plsc.store_scatter(list_t, [pos // CHUNK, pos % CHUNK],
                           b * 4096 + v, mask=keep)
        plsc.store_scatter(list_j, [pos // CHUNK, pos % CHUNK],
                           b * IDX_PER_BATCH + jl, mask=keep)
        cnt = cnt + jnp.sum(mi)

    n = cnt
    ceil = ((n + CHUNK - 1) // CHUNK) * CHUNK
    last_i = jnp.maximum(n - 1, 0)
    lt = plsc.load_gather(
        list_t, [jnp.full((L,), last_i // CHUNK, jnp.int32),
                 jnp.full((L,), last_i % CHUNK, jnp.int32)])
    lj = plsc.load_gather(
        list_j, [jnp.full((L,), last_i // CHUNK, jnp.int32),
                 jnp.full((L,), last_i % CHUNK, jnp.int32)])
    for k in range(CHUNK // L):
        pos2 = n + k * L + iota
        m = pos2 < ceil
        pc = jnp.clip(pos2, 0, IDX_PER_BATCH - 1)
        plsc.store_scatter(list_t, [pc // CHUNK, pc % CHUNK], lt, mask=m)
        plsc.store_scatter(list_j, [pc // CHUNK, pc % CHUNK], lj, mask=m)

    nch = ceil // CHUNK
    NBUF = 3
    bufs = (buf_a, buf_b, buf_c)
    semg = (semg0, semg1, semg2)
    semp = (semp0, semp1, semp2)

    def g_desc(k):
        return pltpu.make_async_copy(img_hbm.at[list_j.at[k]],
                                     bufs[k % NBUF], semg[k % NBUF])

    def p_desc(k):
        return pltpu.make_async_copy(bufs[k % NBUF],
                                     out_hbm.at[list_t.at[k]], semp[k % NBUF])

    for k in range(NCH + 1):
        if k < NCH:
            @pl.when(k < nch)
            def _(k=k):
                if k >= NBUF:
                    p_desc(k - NBUF).wait()
                g_desc(k).start()
        if k >= 1:
            @pl.when(k - 1 < nch)
            def _(k=k):
                g_desc(k - 1).wait()
                p_desc(k - 1).start()
    for k in range(NCH):
        @pl.when((k < nch) & (k + NBUF >= nch))
        def _(k=k):
            p_desc(k).wait()


@jax.jit
def _interleave(img_flat, text_flat, vi_flat):
    nrows, d = text_flat.shape
    br = 512
    copied = pl.pallas_call(
        _copy_body,
        grid=(nrows // br,),
        in_specs=[pl.BlockSpec((br, d), lambda i: (i, 0))],
        out_specs=pl.BlockSpec((br, d), lambda i: (i, 0)),
        out_shape=jax.ShapeDtypeStruct((nrows, d), text_flat.dtype),
    )(text_flat)

    mesh = plsc.VectorSubcoreMesh(core_axis_name="c", subcore_axis_name="s")
    kern = pl_mpmd._mpmd_map(
        [(mesh, _sc_body)],
        jax.ShapeDtypeStruct((nrows, d), text_flat.dtype),
        input_output_aliases={1: 0},
        scratch_types=[
            pltpu.VMEM((IDX_PER_BATCH,), jnp.int32),
            pltpu.VMEM((ROWS_PER_TILE,), jnp.int32),
            pltpu.VMEM((NCH, CHUNK), jnp.int32),
            pltpu.VMEM((NCH, CHUNK), jnp.int32),
            pltpu.VMEM((CHUNK, 2048), jnp.float32),
            pltpu.VMEM((CHUNK, 2048), jnp.float32),
            pltpu.VMEM((CHUNK, 2048), jnp.float32),
            pltpu.SemaphoreType.DMA,
            pltpu.SemaphoreType.DMA,
            pltpu.SemaphoreType.DMA,
            pltpu.SemaphoreType.DMA,
            pltpu.SemaphoreType.DMA,
            pltpu.SemaphoreType.DMA,
        ],
        compiler_params=pltpu.CompilerParams(needs_layout_passes=False),
    )
    return kern(img_flat, copied, vi_flat)


def kernel(image_embeddings, text_embeddings, vision_indices):
    B, S, D = text_embeddings.shape
    img_flat = image_embeddings.reshape(-1, D)
    text_flat = text_embeddings.reshape(B * S, D)
    vi_flat = vision_indices.astype(jnp.int32).reshape(-1)
    out = _interleave(img_flat, text_flat, vi_flat)
    return out.reshape(B, S, D)


# speedup vs baseline: 1.2517x; 1.0149x over previous
import jax
import jax.numpy as jnp
from jax import lax
from jax.experimental import pallas as pl
from jax.experimental.pallas import tpu as pltpu
from jax.experimental.pallas import tpu_sc as plsc
from jax._src.pallas import mpmd as pl_mpmd

L = 16
ROWS_PER_TILE = 512
IDX_PER_BATCH = 512
CHUNK = 16
NCH = IDX_PER_BATCH // CHUNK


def _copy_body(t_ref, o_ref):
    o_ref[...] = t_ref[...]


def _sc_body(img_hbm, copied_hbm, vi_hbm, out_hbm,
             idx_v, winner, list_t, list_j, buf_a, buf_b, buf_c,
             semg0, semg1, semg2, semp0, semp1, semp2):
    del copied_hbm
    nc = 2
    wid = lax.axis_index("s") * nc + lax.axis_index("c")
    b = wid // 8
    seg = wid % 8
    seg_lo = seg * ROWS_PER_TILE

    pltpu.sync_copy(vi_hbm.at[pl.ds(b * IDX_PER_BATCH, IDX_PER_BATCH)], idx_v)

    iota = lax.iota(jnp.int32, L)

    shift1 = jnp.minimum(iota + 1, L - 1)
    for c in range(IDX_PER_BATCH // L):
        v = idx_v[pl.ds(c * L, L)]
        jl = iota + c * L
        ks, js = plsc.sort_key_val(v * IDX_PER_BATCH + jl, jl)
        vs = ks // IDX_PER_BATCH
        nxt = vs.at[shift1].get(mode="promise_in_bounds")
        is_last = (vs != nxt) | (iota == L - 1)
        valid = ((vs >= seg_lo) & (vs < seg_lo + ROWS_PER_TILE)
                 & (vs != 0) & is_last)
        addr = jnp.clip(vs - seg_lo, 0, ROWS_PER_TILE - 1)
        plsc.store_scatter(winner, [addr], js, mask=valid)

    cnt = jnp.int32(0)
    for c in range(IDX_PER_BATCH // L):
        v = idx_v[pl.ds(c * L, L)]
        jl = iota + c * L
        valid0 = (v >= seg_lo) & (v < seg_lo + ROWS_PER_TILE) & (v != 0)
        addr = jnp.clip(v - seg_lo, 0, ROWS_PER_TILE - 1)
        w = plsc.load_gather(winner, [addr], mask=valid0)
        keep = valid0 & (w == jl)
        mi = keep.astype(jnp.int32)
        incl = plsc.cumsum(mi)
        pos = cnt + incl - mi
        plsc.store_scatter(list_t, [pos // CHUNK, pos % CHUNK],
                           b * 4096 + v, mask=keep)
        plsc.store_scatter(list_j, [pos // CHUNK, pos % CHUNK],
                           b * IDX_PER_BATCH + jl, mask=keep)
        cnt = cnt + jnp.sum(mi)

    n = cnt
    ceil = ((n + CHUNK - 1) // CHUNK) * CHUNK
    last_i = jnp.maximum(n - 1, 0)
    lt = plsc.load_gather(
        list_t, [jnp.full((L,), last_i // CHUNK, jnp.int32),
                 jnp.full((L,), last_i % CHUNK, jnp.int32)])
    lj = plsc.load_gather(
        list_j, [jnp.full((L,), last_i // CHUNK, jnp.int32),
                 jnp.full((L,), last_i % CHUNK, jnp.int32)])
    for k in range(CHUNK // L):
        pos2 = n + k * L + iota
        m = pos2 < ceil
        pc = jnp.clip(pos2, 0, IDX_PER_BATCH - 1)
        plsc.store_scatter(list_t, [pc // CHUNK, pc % CHUNK], lt, mask=m)
        plsc.store_scatter(list_j, [pc // CHUNK, pc % CHUNK], lj, mask=m)

    nch = ceil // CHUNK
    NBUF = 3
    bufs = (buf_a, buf_b, buf_c)
    semg = (semg0, semg1, semg2)
    semp = (semp0, semp1, semp2)

    def g_desc(k):
        return pltpu.make_async_copy(img_hbm.at[list_j.at[k]],
                                     bufs[k % NBUF], semg[k % NBUF])

    def p_desc(k):
        return pltpu.make_async_copy(bufs[k % NBUF],
                                     out_hbm.at[list_t.at[k]], semp[k % NBUF])

    for k in range(NCH + 1):
        if k < NCH:
            @pl.when(k < nch)
            def _(k=k):
                if k >= NBUF:
                    p_desc(k - NBUF).wait()
                g_desc(k).start()
        if k >= 1:
            @pl.when(k - 1 < nch)
            def _(k=k):
                g_desc(k - 1).wait()
                p_desc(k - 1).start()
    for k in range(NCH):
        @pl.when((k < nch) & (k + NBUF >= nch))
        def _(k=k):
            p_desc(k).wait()


@jax.jit
def _interleave(img_flat, text_flat, vi_flat):
    nrows, d = text_flat.shape
    br = 1024
    copied = pl.pallas_call(
        _copy_body,
        grid=(nrows // br,),
        in_specs=[pl.BlockSpec((br, d), lambda i: (i, 0))],
        out_specs=pl.BlockSpec((br, d), lambda i: (i, 0)),
        out_shape=jax.ShapeDtypeStruct((nrows, d), text_flat.dtype),
    )(text_flat)

    mesh = plsc.VectorSubcoreMesh(core_axis_name="c", subcore_axis_name="s")
    kern = pl_mpmd._mpmd_map(
        [(mesh, _sc_body)],
        jax.ShapeDtypeStruct((nrows, d), text_flat.dtype),
        input_output_aliases={1: 0},
        scratch_types=[
            pltpu.VMEM((IDX_PER_BATCH,), jnp.int32),
            pltpu.VMEM((ROWS_PER_TILE,), jnp.int32),
            pltpu.VMEM((NCH, CHUNK), jnp.int32),
            pltpu.VMEM((NCH, CHUNK), jnp.int32),
            pltpu.VMEM((CHUNK, 2048), jnp.float32),
            pltpu.VMEM((CHUNK, 2048), jnp.float32),
            pltpu.VMEM((CHUNK, 2048), jnp.float32),
            pltpu.SemaphoreType.DMA,
            pltpu.SemaphoreType.DMA,
            pltpu.SemaphoreType.DMA,
            pltpu.SemaphoreType.DMA,
            pltpu.SemaphoreType.DMA,
            pltpu.SemaphoreType.DMA,
        ],
        compiler_params=pltpu.CompilerParams(needs_layout_passes=False),
    )
    return kern(img_flat, copied, vi_flat)


def kernel(image_embeddings, text_embeddings, vision_indices):
    B, S, D = text_embeddings.shape
    img_flat = image_embeddings.reshape(-1, D)
    text_flat = text_embeddings.reshape(B * S, D)
    vi_flat = vision_indices.astype(jnp.int32).reshape(-1)
    out = _interleave(img_flat, text_flat, vi_flat)
    return out.reshape(B, S, D)
